# baseline (speedup 1.0000x reference)
import functools

import jax
import jax.numpy as jnp
from jax import lax
from jax.experimental import pallas as pl
from jax.experimental.pallas import tpu as pltpu
from jax.experimental.pallas import tpu_sc as plsc

BATCH = 16384
D = 32
SUP = 128
NROWS = 1000000
QP = 262144
NC = 2
NS = 16
NW = NC * NS
BPW = BATCH // NW
CHUNK = 128
NCHUNK = BPW // CHUNK


TB = 8192
TGRID = QP // TB
NB = (NROWS + TB - 1) // TB


def _transpose_body(a0_ref, a1_ref, a2_ref, a3_ref, eye_ref, out_ref):
    acc = None
    for c, a_ref in enumerate((a0_ref, a1_ref, a2_ref, a3_ref)):
        part = lax.dot_general(
            a_ref[...], eye_ref[c * D:(c + 1) * D, :],
            (((0,), (0,)), ((), ())), preferred_element_type=jnp.float32)
        acc = part if acc is None else acc + part
    out_ref[...] = acc


def _to_super(table_t):
    return pl.pallas_call(
        _transpose_body,
        grid=(TGRID,),
        in_specs=[
            pl.BlockSpec((D, TB),
                         lambda i, q=q: (0, jnp.minimum(q * TGRID + i, NB - 1)))
            for q in range(4)
        ] + [pl.BlockSpec((SUP, SUP), lambda i: (0, 0))],
        out_specs=pl.BlockSpec((TB, SUP), lambda i: (i, 0)),
        out_shape=jax.ShapeDtypeStruct((QP, SUP), jnp.float32),
    )(table_t, table_t, table_t, table_t, jnp.eye(SUP, dtype=jnp.float32))


_sc_mesh = plsc.VectorSubcoreMesh(core_axis_name="c", subcore_axis_name="s")

RPW = QP // NW
NBLK = RPW // CHUNK
PHYS_COLS = 1000064
OUTW = CHUNK * SUP


@functools.partial(
    pl.kernel,
    mesh=_sc_mesh,
    out_type=jax.ShapeDtypeStruct((QP * SUP,), jnp.float32),
    scratch_types=[
        pltpu.VMEM((2, 4, D, CHUNK), jnp.float32),
        pltpu.VMEM((2 * OUTW,), jnp.float32),
        pltpu.SemaphoreType.DMA,
        pltpu.SemaphoreType.DMA,
        pltpu.SemaphoreType.DMA,
    ],
    compiler_params=pltpu.CompilerParams(needs_layout_passes=False),
)
def _sc_to_super(tab_hbm, out_hbm, in_v, out_v, sin0, sin1, sout):
    wid = lax.axis_index("s") * NC + lax.axis_index("c")
    wbase = wid * RPW
    sins = (sin0, sin1)
    iota = lax.iota(jnp.int32, 16)

    def fire(b, slot):
        sup0 = wbase + b * CHUNK
        cps = []
        for q in range(4):
            c0 = jnp.minimum(q * QP + sup0, PHYS_COLS - CHUNK)
            cps.append(pltpu.async_copy(
                tab_hbm.at[pl.ds(0, D), pl.ds(c0, CHUNK)],
                in_v.at[slot, q], sins[slot]))
        return cps

    def drain(cps):
        for cp in cps:
            cp.wait()

    out_cps = [None, None]
    cps_next = fire(0, 0)
    for b in range(NBLK):
        slot = b & 1
        cps_cur = cps_next
        cps_next = fire(b + 1, (b + 1) & 1) if b + 1 < NBLK else []
        drain(cps_cur)
        if out_cps[slot] is not None:
            out_cps[slot].wait()
        obase = slot * OUTW
        ss = jnp.full((16,), slot, jnp.int32)

        def body(r, carry, slot=slot, obase=obase, ss=ss):
            rs = jnp.full((16,), r, jnp.int32)
            sbase = obase + r * SUP
            for q in range(4):
                qs = jnp.full((16,), q, jnp.int32)
                v0 = plsc.load_gather(in_v, [ss, qs, iota, rs])
                v1 = plsc.load_gather(in_v, [ss, qs, iota + 16, rs])
                plsc.store_scatter(out_v, [sbase + q * D + iota], v0)
                plsc.store_scatter(out_v, [sbase + q * D + 16 + iota], v1)
            return carry

        lax.fori_loop(0, CHUNK, body, 0)
        out_cps[slot] = pltpu.async_copy(
            out_v.at[pl.ds(obase, OUTW)],
            out_hbm.at[pl.ds((wbase + b * CHUNK) * SUP, OUTW)], sout)
    drain([c for c in out_cps if c is not None])


@functools.partial(
    pl.kernel,
    mesh=_sc_mesh,
    out_type=jax.ShapeDtypeStruct((BATCH, SUP), jnp.float32),
    scratch_types=[
        pltpu.VMEM((NCHUNK, CHUNK), jnp.int32),
        pltpu.VMEM((2, CHUNK, SUP), jnp.float32),
        pltpu.SemaphoreType.DMA,
        pltpu.SemaphoreType.DMA,
    ],
)
def _sc_gather(idx_hbm, tab_hbm, out_hbm, idx_v, buf_v, s0, s1):
    wid = lax.axis_index("s") * NC + lax.axis_index("c")
    base = wid * BPW
    sems = (s0, s1)
    pltpu.sync_copy(idx_hbm.at[wid], idx_v)
    cps = [None] * NCHUNK
    cps[0] = pltpu.async_copy(tab_hbm.at[idx_v.at[0]], buf_v.at[0], sems[0])
    for c in range(NCHUNK):
        s, n = c & 1, (c + 1) & 1
        if c + 1 < NCHUNK:
            cps[c + 1] = pltpu.async_copy(
                tab_hbm.at[idx_v.at[c + 1]], buf_v.at[n], sems[n])
        cps[c].wait()
        pltpu.sync_copy(buf_v.at[s], out_hbm.at[pl.ds(base + c * CHUNK, CHUNK)])


BB = 2048
NPAD = 8


def _select_subrow(raw, off):
    x = raw[:, 0:D]
    for c in range(1, 4):
        x = jnp.where(off == c, raw[:, c * D:(c + 1) * D], x)
    return x


def _mlp_body(ue_ref, ie_ref, uoff_ref, ioff_ref, w1u_ref, w1i_ref, b1_ref,
              w2_ref, b2_ref, out_ref):
    xu = _select_subrow(ue_ref[...], uoff_ref[...])
    xi = _select_subrow(ie_ref[...], ioff_ref[...])
    h = jnp.dot(xu, w1u_ref[...], preferred_element_type=jnp.float32)
    h = h + jnp.dot(xi, w1i_ref[...], preferred_element_type=jnp.float32)
    h = jnp.maximum(h + b1_ref[...], 0.0)
    logits = jnp.dot(h, w2_ref[...], preferred_element_type=jnp.float32) + b2_ref[...]
    lane = lax.broadcasted_iota(jnp.int32, logits.shape, 1)
    masked = jnp.where(lane < 5, logits, -jnp.inf)
    m = jnp.max(masked, axis=1, keepdims=True)
    e = jnp.exp(masked - m)
    s = jnp.sum(e, axis=1, keepdims=True)
    out_ref[...] = (e / s)[:, :5]


def _mlp(ue, ie, uoff, ioff, w1u, w1i, b1, w2p, b2p):
    grid = (BATCH // BB,)
    return pl.pallas_call(
        _mlp_body,
        grid=grid,
        in_specs=[
            pl.BlockSpec((BB, SUP), lambda i: (i, 0)),
            pl.BlockSpec((BB, SUP), lambda i: (i, 0)),
            pl.BlockSpec((BB, 1), lambda i: (i, 0)),
            pl.BlockSpec((BB, 1), lambda i: (i, 0)),
            pl.BlockSpec((D, 64), lambda i: (0, 0)),
            pl.BlockSpec((D, 64), lambda i: (0, 0)),
            pl.BlockSpec((1, 64), lambda i: (0, 0)),
            pl.BlockSpec((64, NPAD), lambda i: (0, 0)),
            pl.BlockSpec((1, NPAD), lambda i: (0, 0)),
        ],
        out_specs=pl.BlockSpec((BB, 5), lambda i: (i, 0)),
        out_shape=jax.ShapeDtypeStruct((BATCH, 5), jnp.float32),
    )(ue, ie, uoff, ioff, w1u, w1i, b1, w2p, b2p)


def kernel(user_ids, item_ids, user_table, item_table, W1, b1, W2, b2):
    uid = user_ids.astype(jnp.int32)
    iid = item_ids.astype(jnp.int32)
    usup = jnp.reshape(uid % QP, (NW, NCHUNK, CHUNK))
    isup = jnp.reshape(iid % QP, (NW, NCHUNK, CHUNK))
    it128 = jnp.reshape(_sc_to_super(jnp.transpose(item_table)), (QP, SUP))
    ut128 = _to_super(jnp.transpose(user_table))
    ue = _sc_gather(usup, ut128)
    ie = _sc_gather(isup, it128)
    uoff = jnp.reshape(uid // QP, (BATCH, 1))
    ioff = jnp.reshape(iid // QP, (BATCH, 1))
    w1u = jnp.transpose(W1[:, :D])
    w1i = jnp.transpose(W1[:, D:])
    w2p = jnp.pad(jnp.transpose(W2), ((0, 0), (0, NPAD - 5)))
    b2p = jnp.pad(jnp.reshape(b2, (1, 5)), ((0, 0), (0, NPAD - 5)))
    return _mlp(ue, ie, uoff, ioff, w1u, w1i, jnp.reshape(b1, (1, 64)), w2p, b2p)


# speedup vs baseline: 2.0930x; 2.0930x over previous
import functools

import jax
import jax.numpy as jnp
from jax import lax
from jax.experimental import pallas as pl
from jax.experimental.pallas import tpu as pltpu
from jax.experimental.pallas import tpu_sc as plsc

BATCH = 16384
D = 32
SUP = 128
NROWS = 1000000
QP = 262144
NC = 2
NS = 16
NW = NC * NS
BPW = BATCH // NW
CHUNK = 128
NCHUNK = BPW // CHUNK


TB = 8192
TGRID = QP // TB
NB = (NROWS + TB - 1) // TB


def _transpose_body(a0_ref, a1_ref, a2_ref, a3_ref, eye_ref, out_ref):
    acc = None
    for c, a_ref in enumerate((a0_ref, a1_ref, a2_ref, a3_ref)):
        part = lax.dot_general(
            a_ref[...], eye_ref[c * D:(c + 1) * D, :],
            (((0,), (0,)), ((), ())), preferred_element_type=jnp.float32)
        acc = part if acc is None else acc + part
    out_ref[...] = acc


def _to_super(table_t):
    return pl.pallas_call(
        _transpose_body,
        grid=(TGRID,),
        in_specs=[
            pl.BlockSpec((D, TB),
                         lambda i, q=q: (0, jnp.minimum(q * TGRID + i, NB - 1)))
            for q in range(4)
        ] + [pl.BlockSpec((SUP, SUP), lambda i: (0, 0))],
        out_specs=pl.BlockSpec((TB, SUP), lambda i: (i, 0)),
        out_shape=jax.ShapeDtypeStruct((QP, SUP), jnp.float32),
    )(table_t, table_t, table_t, table_t, jnp.eye(SUP, dtype=jnp.float32))


_sc_mesh = plsc.VectorSubcoreMesh(core_axis_name="c", subcore_axis_name="s")


@functools.partial(
    pl.kernel,
    mesh=_sc_mesh,
    out_type=jax.ShapeDtypeStruct((BATCH, SUP), jnp.float32),
    scratch_types=[
        pltpu.VMEM((NCHUNK, CHUNK), jnp.int32),
        pltpu.VMEM((2, CHUNK, SUP), jnp.float32),
        pltpu.SemaphoreType.DMA,
        pltpu.SemaphoreType.DMA,
    ],
)
def _sc_gather(idx_hbm, tab_hbm, out_hbm, idx_v, buf_v, s0, s1):
    wid = lax.axis_index("s") * NC + lax.axis_index("c")
    base = wid * BPW
    sems = (s0, s1)
    pltpu.sync_copy(idx_hbm.at[wid], idx_v)
    cps = [None] * NCHUNK
    cps[0] = pltpu.async_copy(tab_hbm.at[idx_v.at[0]], buf_v.at[0], sems[0])
    for c in range(NCHUNK):
        s, n = c & 1, (c + 1) & 1
        if c + 1 < NCHUNK:
            cps[c + 1] = pltpu.async_copy(
                tab_hbm.at[idx_v.at[c + 1]], buf_v.at[n], sems[n])
        cps[c].wait()
        pltpu.sync_copy(buf_v.at[s], out_hbm.at[pl.ds(base + c * CHUNK, CHUNK)])


BB = 2048
NPAD = 8


def _select_subrow(raw, off):
    x = raw[:, 0:D]
    for c in range(1, 4):
        x = jnp.where(off == c, raw[:, c * D:(c + 1) * D], x)
    return x


def _mlp_body(ue_ref, ie_ref, uoff_ref, ioff_ref, w1u_ref, w1i_ref, b1_ref,
              w2_ref, b2_ref, out_ref):
    xu = _select_subrow(ue_ref[...], uoff_ref[...])
    xi = _select_subrow(ie_ref[...], ioff_ref[...])
    h = jnp.dot(xu, w1u_ref[...], preferred_element_type=jnp.float32)
    h = h + jnp.dot(xi, w1i_ref[...], preferred_element_type=jnp.float32)
    h = jnp.maximum(h + b1_ref[...], 0.0)
    logits = jnp.dot(h, w2_ref[...], preferred_element_type=jnp.float32) + b2_ref[...]
    lane = lax.broadcasted_iota(jnp.int32, logits.shape, 1)
    masked = jnp.where(lane < 5, logits, -jnp.inf)
    m = jnp.max(masked, axis=1, keepdims=True)
    e = jnp.exp(masked - m)
    s = jnp.sum(e, axis=1, keepdims=True)
    out_ref[...] = (e / s)[:, :5]


def _mlp(ue, ie, uoff, ioff, w1u, w1i, b1, w2p, b2p):
    grid = (BATCH // BB,)
    return pl.pallas_call(
        _mlp_body,
        grid=grid,
        in_specs=[
            pl.BlockSpec((BB, SUP), lambda i: (i, 0)),
            pl.BlockSpec((BB, SUP), lambda i: (i, 0)),
            pl.BlockSpec((BB, 1), lambda i: (i, 0)),
            pl.BlockSpec((BB, 1), lambda i: (i, 0)),
            pl.BlockSpec((D, 64), lambda i: (0, 0)),
            pl.BlockSpec((D, 64), lambda i: (0, 0)),
            pl.BlockSpec((1, 64), lambda i: (0, 0)),
            pl.BlockSpec((64, NPAD), lambda i: (0, 0)),
            pl.BlockSpec((1, NPAD), lambda i: (0, 0)),
        ],
        out_specs=pl.BlockSpec((BB, 5), lambda i: (i, 0)),
        out_shape=jax.ShapeDtypeStruct((BATCH, 5), jnp.float32),
    )(ue, ie, uoff, ioff, w1u, w1i, b1, w2p, b2p)


def kernel(user_ids, item_ids, user_table, item_table, W1, b1, W2, b2):
    uid = user_ids.astype(jnp.int32)
    iid = item_ids.astype(jnp.int32)
    usup = jnp.reshape(uid % QP, (NW, NCHUNK, CHUNK))
    isup = jnp.reshape(iid % QP, (NW, NCHUNK, CHUNK))
    ut128 = _to_super(jnp.transpose(user_table))
    ue = _sc_gather(usup, ut128)
    it128 = _to_super(jnp.transpose(item_table))
    ie = _sc_gather(isup, it128)
    uoff = jnp.reshape(uid // QP, (BATCH, 1))
    ioff = jnp.reshape(iid // QP, (BATCH, 1))
    w1u = jnp.transpose(W1[:, :D])
    w1i = jnp.transpose(W1[:, D:])
    w2p = jnp.pad(jnp.transpose(W2), ((0, 0), (0, NPAD - 5)))
    b2p = jnp.pad(jnp.reshape(b2, (1, 5)), ((0, 0), (0, NPAD - 5)))
    return _mlp(ue, ie, uoff, ioff, w1u, w1i, jnp.reshape(b1, (1, 64)), w2p, b2p)


# speedup vs baseline: 2.1156x; 1.0108x over previous
import functools

import jax
import jax.numpy as jnp
from jax import lax
from jax.experimental import pallas as pl
from jax.experimental.pallas import tpu as pltpu
from jax.experimental.pallas import tpu_sc as plsc

BATCH = 16384
D = 32
SUP = 128
NROWS = 1000000
QP = 262144
NC = 2
NS = 16
NW = NC * NS
BPW = BATCH // NW
CHUNK = 128
NCHUNK = BPW // CHUNK


TB = 16384
TGRID = QP // TB
NB = (NROWS + TB - 1) // TB


def _transpose_body(a0_ref, a1_ref, a2_ref, a3_ref, eye_ref, out_ref):
    acc = None
    for c, a_ref in enumerate((a0_ref, a1_ref, a2_ref, a3_ref)):
        part = lax.dot_general(
            a_ref[...], eye_ref[c * D:(c + 1) * D, :],
            (((0,), (0,)), ((), ())), preferred_element_type=jnp.float32)
        acc = part if acc is None else acc + part
    out_ref[...] = acc


def _to_super(table_t):
    return pl.pallas_call(
        _transpose_body,
        grid=(TGRID,),
        in_specs=[
            pl.BlockSpec((D, TB),
                         lambda i, q=q: (0, jnp.minimum(q * TGRID + i, NB - 1)))
            for q in range(4)
        ] + [pl.BlockSpec((SUP, SUP), lambda i: (0, 0))],
        out_specs=pl.BlockSpec((TB, SUP), lambda i: (i, 0)),
        out_shape=jax.ShapeDtypeStruct((QP, SUP), jnp.float32),
    )(table_t, table_t, table_t, table_t, jnp.eye(SUP, dtype=jnp.float32))


_sc_mesh = plsc.VectorSubcoreMesh(core_axis_name="c", subcore_axis_name="s")


@functools.partial(
    pl.kernel,
    mesh=_sc_mesh,
    out_type=jax.ShapeDtypeStruct((BATCH, SUP), jnp.float32),
    scratch_types=[
        pltpu.VMEM((NCHUNK, CHUNK), jnp.int32),
        pltpu.VMEM((2, CHUNK, SUP), jnp.float32),
        pltpu.SemaphoreType.DMA,
        pltpu.SemaphoreType.DMA,
    ],
)
def _sc_gather(idx_hbm, tab_hbm, out_hbm, idx_v, buf_v, s0, s1):
    wid = lax.axis_index("s") * NC + lax.axis_index("c")
    base = wid * BPW
    sems = (s0, s1)
    pltpu.sync_copy(idx_hbm.at[wid], idx_v)
    cps = [None] * NCHUNK
    cps[0] = pltpu.async_copy(tab_hbm.at[idx_v.at[0]], buf_v.at[0], sems[0])
    for c in range(NCHUNK):
        s, n = c & 1, (c + 1) & 1
        if c + 1 < NCHUNK:
            cps[c + 1] = pltpu.async_copy(
                tab_hbm.at[idx_v.at[c + 1]], buf_v.at[n], sems[n])
        cps[c].wait()
        pltpu.sync_copy(buf_v.at[s], out_hbm.at[pl.ds(base + c * CHUNK, CHUNK)])


BB = 2048
NPAD = 8


def _select_subrow(raw, off):
    x = raw[:, 0:D]
    for c in range(1, 4):
        x = jnp.where(off == c, raw[:, c * D:(c + 1) * D], x)
    return x


def _mlp_body(ue_ref, ie_ref, uoff_ref, ioff_ref, w1u_ref, w1i_ref, b1_ref,
              w2_ref, b2_ref, out_ref):
    xu = _select_subrow(ue_ref[...], uoff_ref[...])
    xi = _select_subrow(ie_ref[...], ioff_ref[...])
    h = jnp.dot(xu, w1u_ref[...], preferred_element_type=jnp.float32)
    h = h + jnp.dot(xi, w1i_ref[...], preferred_element_type=jnp.float32)
    h = jnp.maximum(h + b1_ref[...], 0.0)
    logits = jnp.dot(h, w2_ref[...], preferred_element_type=jnp.float32) + b2_ref[...]
    lane = lax.broadcasted_iota(jnp.int32, logits.shape, 1)
    masked = jnp.where(lane < 5, logits, -jnp.inf)
    m = jnp.max(masked, axis=1, keepdims=True)
    e = jnp.exp(masked - m)
    s = jnp.sum(e, axis=1, keepdims=True)
    out_ref[...] = (e / s)[:, :5]


def _mlp(ue, ie, uoff, ioff, w1u, w1i, b1, w2p, b2p):
    grid = (BATCH // BB,)
    return pl.pallas_call(
        _mlp_body,
        grid=grid,
        in_specs=[
            pl.BlockSpec((BB, SUP), lambda i: (i, 0)),
            pl.BlockSpec((BB, SUP), lambda i: (i, 0)),
            pl.BlockSpec((BB, 1), lambda i: (i, 0)),
            pl.BlockSpec((BB, 1), lambda i: (i, 0)),
            pl.BlockSpec((D, 64), lambda i: (0, 0)),
            pl.BlockSpec((D, 64), lambda i: (0, 0)),
            pl.BlockSpec((1, 64), lambda i: (0, 0)),
            pl.BlockSpec((64, NPAD), lambda i: (0, 0)),
            pl.BlockSpec((1, NPAD), lambda i: (0, 0)),
        ],
        out_specs=pl.BlockSpec((BB, 5), lambda i: (i, 0)),
        out_shape=jax.ShapeDtypeStruct((BATCH, 5), jnp.float32),
    )(ue, ie, uoff, ioff, w1u, w1i, b1, w2p, b2p)


def kernel(user_ids, item_ids, user_table, item_table, W1, b1, W2, b2):
    uid = user_ids.astype(jnp.int32)
    iid = item_ids.astype(jnp.int32)
    usup = jnp.reshape(uid % QP, (NW, NCHUNK, CHUNK))
    isup = jnp.reshape(iid % QP, (NW, NCHUNK, CHUNK))
    ut128 = _to_super(jnp.transpose(user_table))
    ue = _sc_gather(usup, ut128)
    it128 = _to_super(jnp.transpose(item_table))
    ie = _sc_gather(isup, it128)
    uoff = jnp.reshape(uid // QP, (BATCH, 1))
    ioff = jnp.reshape(iid // QP, (BATCH, 1))
    w1u = jnp.transpose(W1[:, :D])
    w1i = jnp.transpose(W1[:, D:])
    w2p = jnp.pad(jnp.transpose(W2), ((0, 0), (0, NPAD - 5)))
    b2p = jnp.pad(jnp.reshape(b2, (1, 5)), ((0, 0), (0, NPAD - 5)))
    return _mlp(ue, ie, uoff, ioff, w1u, w1i, jnp.reshape(b1, (1, 64)), w2p, b2p)


# speedup vs baseline: 2.1162x; 1.0003x over previous
import functools

import jax
import jax.numpy as jnp
from jax import lax
from jax.experimental import pallas as pl
from jax.experimental.pallas import tpu as pltpu
from jax.experimental.pallas import tpu_sc as plsc

BATCH = 16384
D = 32
SUP = 128
NROWS = 1000000
QP = 262144
NC = 2
NS = 16
NW = NC * NS
BPW = BATCH // NW
CHUNK = 128
NCHUNK = BPW // CHUNK


TB = 16384
TGRID = QP // TB
NB = (NROWS + TB - 1) // TB


def _transpose_body(a0_ref, a1_ref, a2_ref, a3_ref, eye_ref, out_ref):
    acc = None
    for c, a_ref in enumerate((a0_ref, a1_ref, a2_ref, a3_ref)):
        part = lax.dot_general(
            a_ref[...], eye_ref[c * D:(c + 1) * D, :],
            (((0,), (0,)), ((), ())), preferred_element_type=jnp.float32)
        acc = part if acc is None else acc + part
    out_ref[...] = acc


def _to_super(table_t):
    return pl.pallas_call(
        _transpose_body,
        grid=(TGRID,),
        in_specs=[
            pl.BlockSpec((D, TB),
                         lambda i, q=q: (0, jnp.minimum(q * TGRID + i, NB - 1)))
            for q in range(4)
        ] + [pl.BlockSpec((SUP, SUP), lambda i: (0, 0))],
        out_specs=pl.BlockSpec((TB, SUP), lambda i: (i, 0)),
        out_shape=jax.ShapeDtypeStruct((QP, SUP), jnp.float32),
    )(table_t, table_t, table_t, table_t, jnp.eye(SUP, dtype=jnp.float32))


_sc_mesh = plsc.VectorSubcoreMesh(core_axis_name="c", subcore_axis_name="s")


@functools.partial(
    pl.kernel,
    mesh=_sc_mesh,
    out_type=jax.ShapeDtypeStruct((BATCH, SUP), jnp.float32),
    scratch_types=[
        pltpu.VMEM((NCHUNK, CHUNK), jnp.int32),
        pltpu.VMEM((2, CHUNK, SUP), jnp.float32),
        pltpu.SemaphoreType.DMA,
        pltpu.SemaphoreType.DMA,
    ],
)
def _sc_gather(idx_hbm, tab_hbm, out_hbm, idx_v, buf_v, s0, s1):
    wid = lax.axis_index("s") * NC + lax.axis_index("c")
    base = wid * BPW
    sems = (s0, s1)
    pltpu.sync_copy(idx_hbm.at[wid], idx_v)
    cps = [None] * NCHUNK
    cps[0] = pltpu.async_copy(tab_hbm.at[idx_v.at[0]], buf_v.at[0], sems[0])
    for c in range(NCHUNK):
        s, n = c & 1, (c + 1) & 1
        if c + 1 < NCHUNK:
            cps[c + 1] = pltpu.async_copy(
                tab_hbm.at[idx_v.at[c + 1]], buf_v.at[n], sems[n])
        cps[c].wait()
        pltpu.sync_copy(buf_v.at[s], out_hbm.at[pl.ds(base + c * CHUNK, CHUNK)])


BB = 2048
NPAD = 8


def _select_subrow(raw, off):
    x = raw[:, 0:D]
    for c in range(1, 4):
        x = jnp.where(off == c, raw[:, c * D:(c + 1) * D], x)
    return x


def _mlp_body(ue_ref, ie_ref, uoff_ref, ioff_ref, w1u_ref, w1i_ref, b1_ref,
              w2_ref, b2_ref, out_ref):
    xu = _select_subrow(ue_ref[...], uoff_ref[...])
    xi = _select_subrow(ie_ref[...], ioff_ref[...])
    h = jnp.dot(xu, w1u_ref[...], preferred_element_type=jnp.float32)
    h = h + jnp.dot(xi, w1i_ref[...], preferred_element_type=jnp.float32)
    h = jnp.maximum(h + b1_ref[...], 0.0)
    logits = jnp.dot(h, w2_ref[...], preferred_element_type=jnp.float32) + b2_ref[...]
    m = jnp.max(logits, axis=1, keepdims=True)
    e = jnp.exp(logits - m)
    s = jnp.sum(e, axis=1, keepdims=True)
    out_ref[...] = (e / s)[:, :5]


def _mlp(ue, ie, uoff, ioff, w1u, w1i, b1, w2p, b2p):
    grid = (BATCH // BB,)
    return pl.pallas_call(
        _mlp_body,
        grid=grid,
        in_specs=[
            pl.BlockSpec((BB, SUP), lambda i: (i, 0)),
            pl.BlockSpec((BB, SUP), lambda i: (i, 0)),
            pl.BlockSpec((BB, 1), lambda i: (i, 0)),
            pl.BlockSpec((BB, 1), lambda i: (i, 0)),
            pl.BlockSpec((D, 64), lambda i: (0, 0)),
            pl.BlockSpec((D, 64), lambda i: (0, 0)),
            pl.BlockSpec((1, 64), lambda i: (0, 0)),
            pl.BlockSpec((64, NPAD), lambda i: (0, 0)),
            pl.BlockSpec((1, NPAD), lambda i: (0, 0)),
        ],
        out_specs=pl.BlockSpec((BB, 5), lambda i: (i, 0)),
        out_shape=jax.ShapeDtypeStruct((BATCH, 5), jnp.float32),
    )(ue, ie, uoff, ioff, w1u, w1i, b1, w2p, b2p)


def kernel(user_ids, item_ids, user_table, item_table, W1, b1, W2, b2):
    uid = user_ids.astype(jnp.int32)
    iid = item_ids.astype(jnp.int32)
    usup = jnp.reshape(uid % QP, (NW, NCHUNK, CHUNK))
    isup = jnp.reshape(iid % QP, (NW, NCHUNK, CHUNK))
    ut128 = _to_super(jnp.transpose(user_table))
    ue = _sc_gather(usup, ut128)
    it128 = _to_super(jnp.transpose(item_table))
    ie = _sc_gather(isup, it128)
    uoff = jnp.reshape(uid // QP, (BATCH, 1))
    ioff = jnp.reshape(iid // QP, (BATCH, 1))
    w1u = jnp.transpose(W1[:, :D])
    w1i = jnp.transpose(W1[:, D:])
    w2p = jnp.pad(jnp.transpose(W2), ((0, 0), (0, NPAD - 5)))
    b2p = jnp.pad(jnp.reshape(b2, (1, 5)), ((0, 0), (0, NPAD - 5)),
                  constant_values=-1e9)
    return _mlp(ue, ie, uoff, ioff, w1u, w1i, jnp.reshape(b1, (1, 64)), w2p, b2p)
